# Initial kernel scaffold; baseline (speedup 1.0000x reference)
import jax
import jax.numpy as jnp
from jax.experimental import pallas as pl


def kernel(center, pos_c, pos_m, neg_c, neg_m, center_table, context_table):
    raise NotImplementedError("write your pallas kernel here")


# speedup vs baseline: 1.1206x; 1.1206x over previous
import functools

import jax
import jax.numpy as jnp
from jax import lax
from jax.experimental import pallas as pl
from jax.experimental.pallas import tpu as pltpu
from jax.experimental.pallas import tpu_sc as plsc

B = 16384
L = 20
D = 64
_f32 = jnp.float32

_NC = 2
_NS = 16
_NW = _NC * _NS
_CB = B // _NW
_NB = 32
_KB = _NB * L
_NBLK = _CB // _NB
_CHUNK = 128
_LANES = 16


def _make_sc_logits():
    mesh = plsc.VectorSubcoreMesh(core_axis_name="c", subcore_axis_name="s")

    @functools.partial(
        pl.kernel,
        mesh=mesh,
        compiler_params=pltpu.CompilerParams(
            needs_layout_passes=False, use_tc_tiling_on_sc=False),
        out_type=(
            jax.ShapeDtypeStruct((B * L,), _f32),
            jax.ShapeDtypeStruct((B * L,), _f32),
        ),
        scratch_types=[
            pltpu.VMEM((_CB,), jnp.int32),
            pltpu.VMEM((_CB, D), _f32),
            pltpu.VMEM((_KB,), jnp.int32),
            pltpu.VMEM((_KB, D), _f32),
            pltpu.VMEM((_KB,), _f32),
            pltpu.SemaphoreType.DMA,
        ],
    )
    def sc_logits(center_hbm, posc_hbm, negc_hbm, ctab_hbm, xtab_hbm,
                  pos_out, neg_out,
                  cidx_v, crows_v, kidx_v, krows_v, klog_v, sem):
        wid = lax.axis_index("s") * _NC + lax.axis_index("c")
        base = wid * _CB

        pltpu.sync_copy(center_hbm.at[pl.ds(base, _CB)], cidx_v)
        cps = [
            pltpu.async_copy(
                ctab_hbm.at[cidx_v.at[pl.ds(j * _CHUNK, _CHUNK)]],
                crows_v.at[pl.ds(j * _CHUNK, _CHUNK)], sem)
            for j in range(_CB // _CHUNK)
        ]
        for cp in cps:
            cp.wait()

        def run_side(idx_hbm, out_hbm):
            def blk_body(blk, carry):
                off = base * L + blk * _KB
                pltpu.sync_copy(idx_hbm.at[pl.ds(off, _KB)], kidx_v)
                gps = [
                    pltpu.async_copy(
                        xtab_hbm.at[kidx_v.at[pl.ds(j * _CHUNK, _CHUNK)]],
                        krows_v.at[pl.ds(j * _CHUNK, _CHUNK)], sem)
                    for j in range(_KB // _CHUNK)
                ]
                for gp in gps:
                    gp.wait()

                iota = jnp.arange(_LANES, dtype=jnp.int32)
                for ib0 in range(0, _NB, _LANES):

                    def g_body(j, alogs):
                        i = ib0 + j
                        bi = blk * _NB + i
                        cv = [crows_v[bi, pl.ds(kk * _LANES, _LANES)]
                              for kk in range(D // _LANES)]
                        lane = iota == j
                        new = []
                        for ll in range(L):
                            r = i * L + ll
                            acc = krows_v[r, pl.ds(0, _LANES)] * cv[0]
                            for kk in range(1, D // _LANES):
                                acc = acc + (
                                    krows_v[r, pl.ds(kk * _LANES, _LANES)]
                                    * cv[kk])
                            s = jnp.sum(acc)
                            new.append(jnp.where(
                                lane, jnp.full((_LANES,), s, _f32),
                                alogs[ll]))
                        return tuple(new)

                    alogs = lax.fori_loop(
                        0, _LANES, g_body,
                        tuple(jnp.zeros((_LANES,), _f32) for _ in range(L)))
                    for ll in range(L):
                        klog_v[pl.ds(ll * _NB + ib0, _LANES)] = alogs[ll]
                pltpu.sync_copy(klog_v, out_hbm.at[pl.ds(off, _KB)])
                return carry

            lax.fori_loop(0, _NBLK, blk_body, 0)

        run_side(posc_hbm, pos_out)
        run_side(negc_hbm, neg_out)

    return sc_logits


_sc_logits = _make_sc_logits()

_RL = (B * L) // 128


def _logsig(x):
    return jnp.where(x > 0, 0.0, x) - jnp.log1p(jnp.exp(-jnp.abs(x)))


def _tc_loss_kernel(p_ref, n_ref, o_ref):
    s = jnp.sum(_logsig(p_ref[...])) + jnp.sum(_logsig(-n_ref[...]))
    o_ref[0, 0] = -s / B


def _tc_loss(pos_log, neg_log):
    return pl.pallas_call(
        _tc_loss_kernel,
        out_shape=jax.ShapeDtypeStruct((1, 1), _f32),
        out_specs=pl.BlockSpec(memory_space=pltpu.SMEM),
    )(pos_log, neg_log)


def kernel(center, pos_c, pos_m, neg_c, neg_m, center_table, context_table):
    del pos_m, neg_m
    pos_log, neg_log = _sc_logits(
        center, pos_c.reshape(-1), neg_c.reshape(-1),
        center_table, context_table)
    out = _tc_loss(pos_log.reshape(_RL, 128), neg_log.reshape(_RL, 128))
    return out[0, 0]
